# baseline (speedup 1.0000x reference)
import functools

import jax
import jax.numpy as jnp
from jax import lax
from jax.experimental import pallas as pl
from jax.experimental.pallas import tpu as pltpu
from jax.experimental.pallas import tpu_sc as plsc

C = 19
NB = 1024
ROW = 2 * NB + 1
HIST = C * ROW
NW = 32
N = 2 * 512 * 512
PPW = N // NW
CH = 1024
NCH = PPW // CH
HW = 512 * 512


def _sc_hist_body(p_hbm, l_hbm, cnt_out, pbuf, lbuf, hcnt):
    cid = lax.axis_index("c")
    sid = lax.axis_index("s")
    wid = sid * 2 + cid
    img = wid // 16
    base = (wid % 16) * PPW

    zeros16 = jnp.zeros((16,), jnp.float32)

    def zero_body(i, _):
        hcnt[pl.ds(i * 16, 16)] = zeros16
        return 0

    lax.fori_loop(0, HIST // 16, zero_body, 0)

    c_lo = lax.iota(jnp.int32, 16)
    c_hi = c_lo + 16
    m_hi = c_hi < C
    c_safe = jnp.where(m_hi, c_hi, c_hi - 16)
    base_lo = c_lo * ROW
    base_hi = c_safe * ROW
    ones = jnp.ones((16,), jnp.float32)
    fscale = jnp.float32(NB)
    fclamp = jnp.float32(2 * NB - 1)

    K = 8

    def gather_batch(i0):
        lab16 = lbuf[pl.ds(i0, 16)]
        gl = [lab16]
        for j in range(K):
            bi = jnp.full((16,), i0 + j, jnp.int32)
            gl.append(plsc.load_gather(pbuf, [c_lo, bi]))
            gl.append(plsc.load_gather(pbuf, [c_safe, bi]))
        return gl

    def compute_scatter(gl):
        lab16 = gl[0]
        outs = []
        for j in range(K):
            p, p2 = gl[1 + 2 * j], gl[2 + 2 * j]
            lab = lax.gather(
                lab16, jnp.full((16, 1), j, jnp.int32),
                dimension_numbers=lax.GatherDimensionNumbers(
                    offset_dims=(), collapsed_slice_dims=(0,),
                    start_index_map=(0,)),
                slice_sizes=(1,),
                mode=lax.GatherScatterMode.PROMISE_IN_BOUNDS)
            ec = jnp.where(c_lo == lab, 2.0 - p, p)
            bk = jnp.minimum(ec * fscale, fclamp).astype(jnp.int32)
            ec2 = jnp.where(c_hi == lab, 2.0 - p2, p2)
            bk2 = jnp.minimum(ec2 * fscale, fclamp).astype(jnp.int32)
            outs.append((base_lo + bk, base_hi + bk2))
        for idx, idx2 in outs:
            plsc.addupdate_scatter(hcnt, [idx], ones)
            plsc.addupdate_scatter(hcnt, [idx2], ones, mask=m_hi)

    def chunk_body(t, _):
        off = base + t * CH
        pltpu.sync_copy(l_hbm.at[img, pl.ds(off, CH)], lbuf.at[pl.ds(0, CH)])
        pltpu.sync_copy(p_hbm.at[img, :, pl.ds(off, CH)],
                        pbuf.at[:, pl.ds(0, CH)])

        def oct_body(k, carry):
            gl_new = gather_batch(k * K)
            compute_scatter(carry)
            return gl_new

        last = lax.fori_loop(1, CH // K, oct_body, gather_batch(0))
        compute_scatter(last)
        return 0

    lax.fori_loop(0, NCH, chunk_body, 0)

    pltpu.sync_copy(hcnt, cnt_out.at[wid])


_sc_hist = functools.partial(
    pl.kernel,
    out_type=jax.ShapeDtypeStruct((NW, HIST), jnp.float32),
    mesh=plsc.VectorSubcoreMesh(core_axis_name="c", subcore_axis_name="s"),
    compiler_params=pltpu.CompilerParams(needs_layout_passes=False),
    scratch_types=[
        pltpu.VMEM((C, CH + 1), jnp.float32),
        pltpu.VMEM((CH + 16,), jnp.int32),
        pltpu.VMEM((HIST,), jnp.float32),
    ],
)(_sc_hist_body)


def _cumsum_last(x):
    k = 1
    while k < x.shape[-1]:
        pad = jnp.zeros(x.shape[:-1] + (k,), x.dtype)
        x = x + jnp.concatenate([pad, x[..., :-k]], axis=-1)
        k *= 2
    return x


def _tc_finish_body(cnt_ref, out_ref):
    cnt = jnp.sum(cnt_ref[...], axis=0)
    n0, n1 = cnt[:, 0, :], cnt[:, 1, :]
    mids = (lax.broadcasted_iota(jnp.int32, (C, NB), 1).astype(jnp.float32)
            + 0.5) / NB
    s0 = mids * n0
    s1 = mids * n1
    tot0 = jnp.sum(n0, axis=1, keepdims=True)
    tot1 = jnp.sum(n1, axis=1, keepdims=True)
    zb = tot0 - _cumsum_last(n0)
    pb = tot1 - _cumsum_last(n1)
    gts = tot1
    u0 = gts + zb
    inter = gts - pb - n1
    fg_term = s1 / jnp.maximum(u0, 1.0)
    bg_term = s0 * inter / jnp.maximum(u0 * (u0 + n0), 1.0)
    losses = jnp.sum(fg_term + bg_term, axis=1, keepdims=True)
    pres = (gts > 0.0).astype(jnp.float32)
    out_ref[0, 0] = jnp.sum(losses * pres) / jnp.maximum(jnp.sum(pres), 1.0)


def _tc_finish(cnt):
    return pl.pallas_call(
        _tc_finish_body,
        out_shape=jax.ShapeDtypeStruct((1, 1), jnp.float32),
        out_specs=pl.BlockSpec(memory_space=pltpu.MemorySpace.SMEM),
    )(cnt)


def kernel(probas, labels):
    p3 = probas.reshape(2, C, HW)
    l2 = labels.astype(jnp.int32).reshape(2, HW)
    cnt = _sc_hist(p3, l2)
    cnt4 = cnt.reshape(NW, C, ROW)[..., : 2 * NB].reshape(NW, C, 2, NB)
    return _tc_finish(cnt4)[0, 0]


# speedup vs baseline: 1.0662x; 1.0662x over previous
import functools

import jax
import jax.numpy as jnp
from jax import lax
from jax.experimental import pallas as pl
from jax.experimental.pallas import tpu as pltpu
from jax.experimental.pallas import tpu_sc as plsc

C = 19
NB = 1024
ROW = 2 * NB + 1
HIST = C * ROW
NW = 32
N = 2 * 512 * 512
PPW = N // NW
CH = 1024
NCH = PPW // CH
HW = 512 * 512


def _sc_hist_body(p_hbm, l_hbm, cnt_out, pbuf0, pbuf1, lbuf0, lbuf1, hcnt,
                  psem, lsem):
    cid = lax.axis_index("c")
    sid = lax.axis_index("s")
    wid = sid * 2 + cid
    img = wid // 16
    base = (wid % 16) * PPW

    zeros16 = jnp.zeros((16,), jnp.float32)

    def zero_body(i, _):
        hcnt[pl.ds(i * 16, 16)] = zeros16
        return 0

    lax.fori_loop(0, HIST // 16, zero_body, 0)

    c_lo = lax.iota(jnp.int32, 16)
    c_hi = c_lo + 16
    m_hi = c_hi < C
    c_safe = jnp.where(m_hi, c_hi, c_hi - 16)
    base_lo = c_lo * ROW
    base_hi = c_safe * ROW
    ones = jnp.ones((16,), jnp.float32)
    fscale = jnp.float32(NB)
    fclamp = jnp.float32(2 * NB - 1)

    K = 8

    def gather_batch(pb, lb, i0):
        lab16 = lb[pl.ds(i0, 16)]
        gl = [lab16]
        for j in range(K):
            bi = jnp.full((16,), i0 + j, jnp.int32)
            gl.append(plsc.load_gather(pb, [c_lo, bi]))
            gl.append(plsc.load_gather(pb, [c_safe, bi]))
        return gl

    def compute_scatter(gl):
        lab16 = gl[0]
        outs = []
        for j in range(K):
            p, p2 = gl[1 + 2 * j], gl[2 + 2 * j]
            lab = lax.gather(
                lab16, jnp.full((16, 1), j, jnp.int32),
                dimension_numbers=lax.GatherDimensionNumbers(
                    offset_dims=(), collapsed_slice_dims=(0,),
                    start_index_map=(0,)),
                slice_sizes=(1,),
                mode=lax.GatherScatterMode.PROMISE_IN_BOUNDS)
            ec = jnp.where(c_lo == lab, 2.0 - p, p)
            bk = jnp.minimum(ec * fscale, fclamp).astype(jnp.int32)
            ec2 = jnp.where(c_hi == lab, 2.0 - p2, p2)
            bk2 = jnp.minimum(ec2 * fscale, fclamp).astype(jnp.int32)
            outs.append((base_lo + bk, base_hi + bk2))
        for idx, idx2 in outs:
            plsc.addupdate_scatter(hcnt, [idx], ones)
            plsc.addupdate_scatter(hcnt, [idx2], ones, mask=m_hi)

    def _copies(t):
        par = t % 2
        off = base + t * CH
        lb = (lbuf0, lbuf1)[par]
        pb = (pbuf0, pbuf1)[par]
        return (
            pltpu.make_async_copy(l_hbm.at[img, pl.ds(off, CH)],
                                  lb.at[pl.ds(0, CH)], lsem.at[par]),
            pltpu.make_async_copy(p_hbm.at[img, :, pl.ds(off, CH)],
                                  pb.at[:, pl.ds(0, CH)], psem.at[par]),
        )

    def dma_start(t):
        for cp in _copies(t):
            cp.start()

    def dma_wait(t):
        for cp in _copies(t):
            cp.wait()

    dma_start(0)
    for t in range(NCH):
        dma_wait(t)
        if t + 1 < NCH:
            dma_start(t + 1)
        pb = (pbuf0, pbuf1)[t % 2]
        lb = (lbuf0, lbuf1)[t % 2]

        def oct_body(k, carry):
            gl_new = gather_batch(pb, lb, k * K)
            compute_scatter(carry)
            return gl_new

        last = lax.fori_loop(1, CH // K, oct_body, gather_batch(pb, lb, 0))
        compute_scatter(last)

    pltpu.sync_copy(hcnt, cnt_out.at[wid])


_sc_hist = functools.partial(
    pl.kernel,
    out_type=jax.ShapeDtypeStruct((NW, HIST), jnp.float32),
    mesh=plsc.VectorSubcoreMesh(core_axis_name="c", subcore_axis_name="s"),
    compiler_params=pltpu.CompilerParams(needs_layout_passes=False),
    scratch_types=[
        pltpu.VMEM((C, CH + 1), jnp.float32),
        pltpu.VMEM((C, CH + 1), jnp.float32),
        pltpu.VMEM((CH + 16,), jnp.int32),
        pltpu.VMEM((CH + 16,), jnp.int32),
        pltpu.VMEM((HIST,), jnp.float32),
        pltpu.SemaphoreType.DMA((2,)),
        pltpu.SemaphoreType.DMA((2,)),
    ],
)(_sc_hist_body)


def _cumsum_last(x):
    k = 1
    while k < x.shape[-1]:
        pad = jnp.zeros(x.shape[:-1] + (k,), x.dtype)
        x = x + jnp.concatenate([pad, x[..., :-k]], axis=-1)
        k *= 2
    return x


def _tc_finish_body(cnt_ref, out_ref):
    cnt = jnp.sum(cnt_ref[...], axis=0)
    n0, n1 = cnt[:, 0, :], cnt[:, 1, :]
    mids = (lax.broadcasted_iota(jnp.int32, (C, NB), 1).astype(jnp.float32)
            + 0.5) / NB
    s0 = mids * n0
    s1 = mids * n1
    tot0 = jnp.sum(n0, axis=1, keepdims=True)
    tot1 = jnp.sum(n1, axis=1, keepdims=True)
    zb = tot0 - _cumsum_last(n0)
    pb = tot1 - _cumsum_last(n1)
    gts = tot1
    u0 = gts + zb
    inter = gts - pb - n1
    fg_term = s1 / jnp.maximum(u0, 1.0)
    bg_term = s0 * inter / jnp.maximum(u0 * (u0 + n0), 1.0)
    losses = jnp.sum(fg_term + bg_term, axis=1, keepdims=True)
    pres = (gts > 0.0).astype(jnp.float32)
    out_ref[0, 0] = jnp.sum(losses * pres) / jnp.maximum(jnp.sum(pres), 1.0)


def _tc_finish(cnt):
    return pl.pallas_call(
        _tc_finish_body,
        out_shape=jax.ShapeDtypeStruct((1, 1), jnp.float32),
        out_specs=pl.BlockSpec(memory_space=pltpu.MemorySpace.SMEM),
    )(cnt)


def kernel(probas, labels):
    p3 = probas.reshape(2, C, HW)
    l2 = labels.astype(jnp.int32).reshape(2, HW)
    cnt = _sc_hist(p3, l2)
    cnt4 = cnt.reshape(NW, C, ROW)[..., : 2 * NB].reshape(NW, C, 2, NB)
    return _tc_finish(cnt4)[0, 0]
